# baseline (speedup 1.0000x reference)
import functools

import jax
import jax.numpy as jnp
from jax.experimental import pallas as pl

TOPK = 8
NEXP = 64
BLK = 1024


SUB = 4


def _topk_part(logits):
    b = logits.shape[0]
    lane = jax.lax.broadcasted_iota(jnp.int32, (b, NEXP), 1)
    val = logits
    top_vals = []
    top_idx = []
    for _ in range(TOPK):
        m = jnp.max(val, axis=-1, keepdims=True)
        li = jnp.min(jnp.where(val == m, lane, NEXP), axis=-1, keepdims=True)
        top_vals.append(m)
        top_idx.append(li)
        val = jnp.where(lane == li, -jnp.inf, val)
    tv = jnp.concatenate(top_vals, axis=1)
    ti = jnp.concatenate(top_idx, axis=1)
    e = jnp.exp(tv - tv[:, :1])
    return e / jnp.sum(e, axis=-1, keepdims=True), ti


def _gate_kernel(x_ref, w_ref, ow_ref, oi_ref):
    w = w_ref[...]
    c = BLK // SUB
    for i in range(SUB):
        sl = pl.ds(i * c, c)
        logits = jax.lax.dot_general(
            x_ref[sl, :], w,
            dimension_numbers=(((1,), (1,)), ((), ())),
            preferred_element_type=jnp.float32,
        )
        ow, oi = _topk_part(logits)
        ow_ref[sl, :] = ow
        oi_ref[sl, :] = oi


@functools.partial(jax.jit, static_argnames=())
def kernel(x, W):
    n, d = x.shape
    grid = (n // BLK,)
    ow, oi = pl.pallas_call(
        _gate_kernel,
        grid=grid,
        in_specs=[
            pl.BlockSpec((BLK, d), lambda i: (i, 0)),
            pl.BlockSpec((NEXP, d), lambda i: (0, 0)),
        ],
        out_specs=[
            pl.BlockSpec((BLK, TOPK), lambda i: (i, 0)),
            pl.BlockSpec((BLK, TOPK), lambda i: (i, 0)),
        ],
        out_shape=[
            jax.ShapeDtypeStruct((n, TOPK), jnp.float32),
            jax.ShapeDtypeStruct((n, TOPK), jnp.int32),
        ],
    )(x, W)
    return ow.astype(x.dtype), oi


# speedup vs baseline: 1.0383x; 1.0383x over previous
import functools

import jax
import jax.numpy as jnp
from jax.experimental import pallas as pl

TOPK = 8
NEXP = 64
BLK = 1024


SUB = 8


def _topk_part(logits):
    b = logits.shape[0]
    lane = jax.lax.broadcasted_iota(jnp.int32, (b, NEXP), 1)
    val = logits
    top_vals = []
    top_idx = []
    for _ in range(TOPK):
        m = jnp.max(val, axis=-1, keepdims=True)
        li = jnp.min(jnp.where(val == m, lane, NEXP), axis=-1, keepdims=True)
        top_vals.append(m)
        top_idx.append(li)
        val = jnp.where(lane == li, -jnp.inf, val)
    tv = jnp.concatenate(top_vals, axis=1)
    ti = jnp.concatenate(top_idx, axis=1)
    e = jnp.exp(tv - tv[:, :1])
    return e / jnp.sum(e, axis=-1, keepdims=True), ti


def _gate_kernel(x_ref, w_ref, ow_ref, oi_ref):
    w = w_ref[...]
    c = BLK // SUB
    for i in range(SUB):
        sl = pl.ds(i * c, c)
        logits = jax.lax.dot_general(
            x_ref[sl, :], w,
            dimension_numbers=(((1,), (1,)), ((), ())),
            preferred_element_type=jnp.float32,
        )
        ow, oi = _topk_part(logits)
        ow_ref[sl, :] = ow
        oi_ref[sl, :] = oi


@functools.partial(jax.jit, static_argnames=())
def kernel(x, W):
    n, d = x.shape
    grid = (n // BLK,)
    ow, oi = pl.pallas_call(
        _gate_kernel,
        grid=grid,
        in_specs=[
            pl.BlockSpec((BLK, d), lambda i: (i, 0)),
            pl.BlockSpec((NEXP, d), lambda i: (0, 0)),
        ],
        out_specs=[
            pl.BlockSpec((BLK, TOPK), lambda i: (i, 0)),
            pl.BlockSpec((BLK, TOPK), lambda i: (i, 0)),
        ],
        out_shape=[
            jax.ShapeDtypeStruct((n, TOPK), jnp.float32),
            jax.ShapeDtypeStruct((n, TOPK), jnp.int32),
        ],
    )(x, W)
    return ow.astype(x.dtype), oi


# speedup vs baseline: 1.1374x; 1.0954x over previous
import functools

import jax
import jax.numpy as jnp
from jax.experimental import pallas as pl

TOPK = 8
NEXP = 64
BLK = 1024


SUB = 8


def _topk_part(logits):
    b = logits.shape[0]
    lane = jax.lax.broadcasted_iota(jnp.int32, (b, NEXP), 1).astype(
        jnp.float32)
    val = logits
    top_vals = []
    top_idx = []
    for _ in range(TOPK):
        m = jnp.max(val, axis=-1, keepdims=True)
        li = jnp.min(jnp.where(val == m, lane, jnp.float32(NEXP)),
                     axis=-1, keepdims=True)
        top_vals.append(m)
        top_idx.append(li)
        val = jnp.where(lane == li, -jnp.inf, val)
    tv = jnp.concatenate(top_vals, axis=1)
    ti = jnp.concatenate(top_idx, axis=1).astype(jnp.int32)
    e = jnp.exp(tv - tv[:, :1])
    return e / jnp.sum(e, axis=-1, keepdims=True), ti


def _gate_kernel(x_ref, w_ref, ow_ref, oi_ref):
    w = w_ref[...]
    c = BLK // SUB
    for i in range(SUB):
        sl = pl.ds(i * c, c)
        logits = jax.lax.dot_general(
            x_ref[sl, :], w,
            dimension_numbers=(((1,), (1,)), ((), ())),
            preferred_element_type=jnp.float32,
        )
        ow, oi = _topk_part(logits)
        ow_ref[sl, :] = ow
        oi_ref[sl, :] = oi


@functools.partial(jax.jit, static_argnames=())
def kernel(x, W):
    n, d = x.shape
    grid = (n // BLK,)
    ow, oi = pl.pallas_call(
        _gate_kernel,
        grid=grid,
        in_specs=[
            pl.BlockSpec((BLK, d), lambda i: (i, 0)),
            pl.BlockSpec((NEXP, d), lambda i: (0, 0)),
        ],
        out_specs=[
            pl.BlockSpec((BLK, TOPK), lambda i: (i, 0)),
            pl.BlockSpec((BLK, TOPK), lambda i: (i, 0)),
        ],
        out_shape=[
            jax.ShapeDtypeStruct((n, TOPK), jnp.float32),
            jax.ShapeDtypeStruct((n, TOPK), jnp.int32),
        ],
    )(x, W)
    return ow.astype(x.dtype), oi


# speedup vs baseline: 1.1477x; 1.0091x over previous
import functools

import jax
import jax.numpy as jnp
from jax.experimental import pallas as pl

TOPK = 8
NEXP = 64
BLK = 1024


SUB = 4


def _topk_part(logits):
    b = logits.shape[0]
    lane = jax.lax.broadcasted_iota(jnp.int32, (b, NEXP), 1).astype(
        jnp.float32)
    val = logits
    top_vals = []
    top_idx = []
    for _ in range(TOPK):
        m = jnp.max(val, axis=-1, keepdims=True)
        li = jnp.min(jnp.where(val == m, lane, jnp.float32(NEXP)),
                     axis=-1, keepdims=True)
        top_vals.append(m)
        top_idx.append(li)
        val = jnp.where(lane == li, -jnp.inf, val)
    tv = jnp.concatenate(top_vals, axis=1)
    ti = jnp.concatenate(top_idx, axis=1).astype(jnp.int32)
    e = jnp.exp(tv - tv[:, :1])
    return e / jnp.sum(e, axis=-1, keepdims=True), ti


def _gate_kernel(x_ref, w_ref, ow_ref, oi_ref):
    w = w_ref[...]
    c = BLK // SUB
    for i in range(SUB):
        sl = pl.ds(i * c, c)
        logits = jax.lax.dot_general(
            x_ref[sl, :], w,
            dimension_numbers=(((1,), (1,)), ((), ())),
            preferred_element_type=jnp.float32,
        )
        ow, oi = _topk_part(logits)
        ow_ref[sl, :] = ow
        oi_ref[sl, :] = oi


@functools.partial(jax.jit, static_argnames=())
def kernel(x, W):
    n, d = x.shape
    grid = (n // BLK,)
    ow, oi = pl.pallas_call(
        _gate_kernel,
        grid=grid,
        in_specs=[
            pl.BlockSpec((BLK, d), lambda i: (i, 0)),
            pl.BlockSpec((NEXP, d), lambda i: (0, 0)),
        ],
        out_specs=[
            pl.BlockSpec((BLK, TOPK), lambda i: (i, 0)),
            pl.BlockSpec((BLK, TOPK), lambda i: (i, 0)),
        ],
        out_shape=[
            jax.ShapeDtypeStruct((n, TOPK), jnp.float32),
            jax.ShapeDtypeStruct((n, TOPK), jnp.int32),
        ],
    )(x, W)
    return ow.astype(x.dtype), oi
